# baseline (speedup 1.0000x reference)
import jax
import jax.numpy as jnp
from jax.experimental import pallas as pl
from jax.experimental.pallas import tpu as pltpu

B, C, H, W = 256, 256, 14, 14
HW = H * W
ROWS_PER_STEP = 8


def _gather_mask_plane(indices, table_i8):
    def body(idx_ref, *refs):
        tables = refs[:ROWS_PER_STEP]
        o_ref = refs[ROWS_PER_STEP]
        i = pl.program_id(0)
        row_iota = jax.lax.broadcasted_iota(jnp.int32, (8, C), 0)
        for j in range(ROWS_PER_STEP):
            r = idx_ref[i * ROWS_PER_STEP + j] % 8
            slab = tables[j][...].astype(jnp.float32)
            row = jnp.sum(jnp.where(row_iota == r, slab, 0.0), axis=0,
                          keepdims=True)
            o_ref[pl.ds(j, 1), :] = row

    def make_spec(j):
        return pl.BlockSpec(
            (8, C), lambda i, idx, j=j: (idx[i * ROWS_PER_STEP + j] // 8, 0)
        )

    grid_spec = pltpu.PrefetchScalarGridSpec(
        num_scalar_prefetch=1,
        grid=(B // ROWS_PER_STEP,),
        in_specs=[make_spec(j) for j in range(ROWS_PER_STEP)],
        out_specs=pl.BlockSpec((ROWS_PER_STEP, C), lambda i, idx: (i, 0)),
    )
    return pl.pallas_call(
        body,
        grid_spec=grid_spec,
        out_shape=jax.ShapeDtypeStruct((B, C), jnp.float32),
    )(indices, *([table_i8] * ROWS_PER_STEP))


def _mask_multiply(x_planes, mask):
    p = 7
    grid = (HW // p,)

    def body(x_ref, m_ref, o_ref):
        o_ref[...] = x_ref[...] * m_ref[...][None, :, :]

    return pl.pallas_call(
        body,
        grid=grid,
        in_specs=[
            pl.BlockSpec((p, B, C), lambda i: (i, 0, 0)),
            pl.BlockSpec((B, C), lambda i: (0, 0)),
        ],
        out_specs=pl.BlockSpec((p, B, C), lambda i: (i, 0, 0)),
        out_shape=jax.ShapeDtypeStruct((HW, B, C), jnp.float32),
    )(x_planes, mask)


def kernel(X, indices, mask_table):
    table_i8 = mask_table.view(jnp.int8)
    mask = _gather_mask_plane(indices, table_i8)
    x_planes = jnp.transpose(X, (2, 3, 0, 1)).reshape(HW, B, C)
    out = _mask_multiply(x_planes, mask)
    return jnp.transpose(out.reshape(H, W, B, C), (2, 3, 0, 1))


# speedup vs baseline: 1.1106x; 1.1106x over previous
import jax
import jax.numpy as jnp
from jax.experimental import pallas as pl
from jax.experimental.pallas import tpu as pltpu

B, C, H, W = 256, 256, 14, 14
HW = H * W
ROWS_PER_STEP = 8


def _gather_mask_plane(indices, table_i8):
    def body(idx_ref, *refs):
        tables = refs[:ROWS_PER_STEP]
        o_ref = refs[ROWS_PER_STEP]
        i = pl.program_id(0)
        row_iota = jax.lax.broadcasted_iota(jnp.int32, (8, C), 0)
        for j in range(ROWS_PER_STEP):
            r = idx_ref[i * ROWS_PER_STEP + j] % 8
            slab = tables[j][...].astype(jnp.float32)
            row = jnp.sum(jnp.where(row_iota == r, slab, 0.0), axis=0,
                          keepdims=True)
            o_ref[pl.ds(j, 1), :] = row

    def make_spec(j):
        return pl.BlockSpec(
            (8, C), lambda i, idx, j=j: (idx[i * ROWS_PER_STEP + j] // 8, 0)
        )

    grid_spec = pltpu.PrefetchScalarGridSpec(
        num_scalar_prefetch=1,
        grid=(B // ROWS_PER_STEP,),
        in_specs=[make_spec(j) for j in range(ROWS_PER_STEP)],
        out_specs=pl.BlockSpec((ROWS_PER_STEP, C), lambda i, idx: (i, 0)),
    )
    return pl.pallas_call(
        body,
        grid_spec=grid_spec,
        out_shape=jax.ShapeDtypeStruct((B, C), jnp.float32),
    )(indices, *([table_i8] * ROWS_PER_STEP))


def _mask_multiply(x_planes, mask):
    p = 49
    grid = (HW // p,)

    def body(x_ref, m_ref, o_ref):
        o_ref[...] = x_ref[...] * m_ref[...][None, :, :]

    return pl.pallas_call(
        body,
        grid=grid,
        in_specs=[
            pl.BlockSpec((p, B, C), lambda i: (i, 0, 0)),
            pl.BlockSpec((B, C), lambda i: (0, 0)),
        ],
        out_specs=pl.BlockSpec((p, B, C), lambda i: (i, 0, 0)),
        out_shape=jax.ShapeDtypeStruct((HW, B, C), jnp.float32),
    )(x_planes, mask)


def kernel(X, indices, mask_table):
    table_i8 = mask_table.view(jnp.int8)
    mask = _gather_mask_plane(indices, table_i8)
    x_planes = jnp.transpose(X, (2, 3, 0, 1)).reshape(HW, B, C)
    out = _mask_multiply(x_planes, mask)
    return jnp.transpose(out.reshape(H, W, B, C), (2, 3, 0, 1))


# speedup vs baseline: 1.2529x; 1.1282x over previous
import jax
import jax.numpy as jnp
from jax.experimental import pallas as pl
from jax.experimental.pallas import tpu as pltpu

B, C, H, W = 256, 256, 14, 14
HW = H * W
ROWS_PER_STEP = 32


def _gather_mask_plane(indices, table_i8):
    def body(idx_ref, *refs):
        tables = refs[:ROWS_PER_STEP]
        o_ref = refs[ROWS_PER_STEP]
        i = pl.program_id(0)
        row_iota = jax.lax.broadcasted_iota(jnp.int32, (8, C), 0)
        for j in range(ROWS_PER_STEP):
            r = idx_ref[i * ROWS_PER_STEP + j] % 8
            slab = tables[j][...].astype(jnp.float32)
            row = jnp.sum(jnp.where(row_iota == r, slab, 0.0), axis=0,
                          keepdims=True)
            o_ref[pl.ds(j, 1), :] = row

    def make_spec(j):
        return pl.BlockSpec(
            (8, C), lambda i, idx, j=j: (idx[i * ROWS_PER_STEP + j] // 8, 0)
        )

    grid_spec = pltpu.PrefetchScalarGridSpec(
        num_scalar_prefetch=1,
        grid=(B // ROWS_PER_STEP,),
        in_specs=[make_spec(j) for j in range(ROWS_PER_STEP)],
        out_specs=pl.BlockSpec((ROWS_PER_STEP, C), lambda i, idx: (i, 0)),
    )
    return pl.pallas_call(
        body,
        grid_spec=grid_spec,
        out_shape=jax.ShapeDtypeStruct((B, C), jnp.float32),
    )(indices, *([table_i8] * ROWS_PER_STEP))


def _mask_multiply(x_planes, mask):
    p = 49
    grid = (HW // p,)

    def body(x_ref, m_ref, o_ref):
        o_ref[...] = x_ref[...] * m_ref[...][None, :, :]

    return pl.pallas_call(
        body,
        grid=grid,
        in_specs=[
            pl.BlockSpec((p, B, C), lambda i: (i, 0, 0)),
            pl.BlockSpec((B, C), lambda i: (0, 0)),
        ],
        out_specs=pl.BlockSpec((p, B, C), lambda i: (i, 0, 0)),
        out_shape=jax.ShapeDtypeStruct((HW, B, C), jnp.float32),
    )(x_planes, mask)


def kernel(X, indices, mask_table):
    table_i8 = mask_table.view(jnp.int8)
    mask = _gather_mask_plane(indices, table_i8)
    x_planes = jnp.transpose(X, (2, 3, 0, 1)).reshape(HW, B, C)
    out = _mask_multiply(x_planes, mask)
    return jnp.transpose(out.reshape(H, W, B, C), (2, 3, 0, 1))


# speedup vs baseline: 1.4243x; 1.1368x over previous
import jax
import jax.numpy as jnp
from jax import lax
from jax.experimental import pallas as pl
from jax.experimental.pallas import tpu as pltpu

B, C, H, W = 256, 256, 14, 14
HW = H * W
MAX_ID = 50000
NS = B * 8


def _fused_mask_multiply(indices, tbl_i8, rsel, x_planes, p):
    grid = (HW // p,)

    def body(idx_ref, x_ref, r_ref, tbl_ref, o_ref, slabs, mask_v, sem):
        step = pl.program_id(0)

        @pl.when(step == 0)
        def _():
            def issue(k, carry):
                base = (idx_ref[k] // 8) * 8
                pltpu.make_async_copy(
                    tbl_ref.at[pl.ds(base, 8), :],
                    slabs.at[pl.ds(k * 8, 8), :],
                    sem,
                ).start()
                return carry

            lax.fori_loop(0, B, issue, 0, unroll=8)

            def drain(k, carry):
                pltpu.make_async_copy(
                    tbl_ref.at[pl.ds(0, 8), :],
                    slabs.at[pl.ds(k * 8, 8), :],
                    sem,
                ).wait()
                return carry

            lax.fori_loop(0, B, drain, 0, unroll=8)

            s = slabs[...].astype(jnp.float32)
            r = r_ref[...]
            ki = lax.broadcasted_iota(jnp.int32, (B, NS), 0)
            ji = lax.broadcasted_iota(jnp.int32, (B, NS), 1)
            sel = (ji == 8 * ki + r).astype(jnp.float32)
            mask_v[...] = jnp.dot(sel, s, preferred_element_type=jnp.float32)

        o_ref[...] = x_ref[...] * mask_v[...][None, :, :]

    grid_spec = pltpu.PrefetchScalarGridSpec(
        num_scalar_prefetch=1,
        grid=grid,
        in_specs=[
            pl.BlockSpec((p, B, C), lambda i, idx: (i, 0, 0)),
            pl.BlockSpec((B, 1), lambda i, idx: (0, 0)),
            pl.BlockSpec(memory_space=pl.ANY),
        ],
        out_specs=pl.BlockSpec((p, B, C), lambda i, idx: (i, 0, 0)),
        scratch_shapes=[
            pltpu.VMEM((NS, C), jnp.int8),
            pltpu.VMEM((B, C), jnp.float32),
            pltpu.SemaphoreType.DMA,
        ],
    )
    return pl.pallas_call(
        body,
        grid_spec=grid_spec,
        out_shape=jax.ShapeDtypeStruct((HW, B, C), jnp.float32),
    )(indices, x_planes, rsel, tbl_i8)


def kernel(X, indices, mask_table):
    tbl_i8 = mask_table.view(jnp.int8)
    rsel = (indices % 8).astype(jnp.int32).reshape(B, 1)
    x_planes = jnp.transpose(X, (2, 3, 0, 1)).reshape(HW, B, C)
    out = _fused_mask_multiply(indices, tbl_i8, rsel, x_planes, p=49)
    return jnp.transpose(out.reshape(H, W, B, C), (2, 3, 0, 1))
